# Initial kernel scaffold; baseline (speedup 1.0000x reference)
import jax
import jax.numpy as jnp
from jax.experimental import pallas as pl


def kernel(class_outs, regression_outs, boxes, image_h, image_w):
    raise NotImplementedError("write your pallas kernel here")


# speedup vs baseline: 1.5426x; 1.5426x over previous
import jax
import jax.numpy as jnp
from jax import lax
from jax.experimental import pallas as pl
from jax.experimental.pallas import tpu as pltpu

_N_PROP = 20000
_N_CLS = 80
_SCORE_THR = 0.05
_IOU_THR = 0.5
_MAX_PER_IMG = 100
_NMS_PRE = 1000
_MAX_RATIO = 4.135166556742356
_OFFSET = 4096.0
_NC = 1024


def _softmax_kernel(x_ref, o_ref):
    x = x_ref[...]
    m = jnp.max(x, axis=1, keepdims=True)
    e = jnp.exp(x - m)
    s = jnp.sum(e, axis=1, keepdims=True)
    o_ref[...] = e[:, :_N_CLS] / s


def _softmax_scores(class_outs):
    blk = 2000
    return pl.pallas_call(
        _softmax_kernel,
        grid=(_N_PROP // blk,),
        in_specs=[pl.BlockSpec((blk, _N_CLS + 1), lambda i: (i, 0))],
        out_specs=pl.BlockSpec((blk, _N_CLS), lambda i: (i, 0)),
        out_shape=jax.ShapeDtypeStruct((_N_PROP, _N_CLS), jnp.float32),
    )(class_outs)


def _decode_rc(rois, deltas, labels, W, H):
    dx = deltas[0] * 0.1
    dy = deltas[1] * 0.1
    dw = jnp.clip(deltas[2] * 0.2, -_MAX_RATIO, _MAX_RATIO)
    dh = jnp.clip(deltas[3] * 0.2, -_MAX_RATIO, _MAX_RATIO)
    pw = rois[2] - rois[0]
    ph = rois[3] - rois[1]
    px = (rois[0] + rois[2]) * 0.5
    py = (rois[1] + rois[3]) * 0.5
    gw = pw * jnp.exp(dw)
    gh = ph * jnp.exp(dh)
    gx = px + pw * dx
    gy = py + ph * dy
    x1 = jnp.clip(gx - gw * 0.5, 0.0, W)
    y1 = jnp.clip(gy - gh * 0.5, 0.0, H)
    x2 = jnp.clip(gx + gw * 0.5, 0.0, W)
    y2 = jnp.clip(gy + gh * 0.5, 0.0, H)
    off = labels * _OFFSET
    return (x1, y1, x2, y2), (x1 + off, y1 + off, x2 + off, y2 + off)


def _smat_kernel(hw_ref, rois_c, deltas_c, labels_c, rois_r, deltas_r,
                 labels_r, s_ref, boxes_c_ref):
    H = hw_ref[0]
    W = hw_ref[1]
    pid = pl.program_id(0)
    roc = rois_c[...]
    dec = deltas_c[...]
    rc = [roc[:, k:k + 1] for k in range(4)]
    dc = [dec[:, k:k + 1] for k in range(4)]
    (x1c, y1c, x2c, y2c), (ox1c, oy1c, ox2c, oy2c) = _decode_rc(
        rc, dc, labels_c[...], W, H)
    boxes_c_ref[:, 0:1] = x1c
    boxes_c_ref[:, 1:2] = y1c
    boxes_c_ref[:, 2:3] = x2c
    boxes_c_ref[:, 3:4] = y2c
    ror = rois_r[...]
    der = deltas_r[...]
    rr = [ror[k:k + 1, :] for k in range(4)]
    dr = [der[k:k + 1, :] for k in range(4)]
    _, (ox1r, oy1r, ox2r, oy2r) = _decode_rc(rr, dr, labels_r[...], W, H)
    area_c = jnp.maximum(ox2c - ox1c, 0.0) * jnp.maximum(oy2c - oy1c, 0.0)
    area_r = jnp.maximum(ox2r - ox1r, 0.0) * jnp.maximum(oy2r - oy1r, 0.0)
    ix1 = jnp.maximum(ox1c, ox1r)
    iy1 = jnp.maximum(oy1c, oy1r)
    ix2 = jnp.minimum(ox2c, ox2r)
    iy2 = jnp.minimum(oy2c, oy2r)
    inter = jnp.maximum(ix2 - ix1, 0.0) * jnp.maximum(iy2 - iy1, 0.0)
    iou = inter / (area_c + area_r - inter + 1e-6)
    gi = pid * 128 + lax.broadcasted_iota(jnp.int32, (128, _NC), 0)
    gj = lax.broadcasted_iota(jnp.int32, (128, _NC), 1)
    s_ref[...] = jnp.where((iou > _IOU_THR) & (gi < gj), 1.0, 0.0)


def _nms_kernel(s_ref, boxes_c_ref, scores_r_ref, scores_c_ref, labels_c_ref,
                x1o, y1o, x2o, y2o, so, lo):
    boxes_c = boxes_c_ref[...]
    scores_r = scores_r_ref[...]
    scores_c = scores_c_ref[...]
    labels_c = labels_c_ref[...]
    iota_r = lax.broadcasted_iota(jnp.int32, (1, _NC), 1)

    def body(i, keep):
        k_i = jnp.sum(jnp.where(iota_r == i, keep, 0.0))
        row = s_ref[pl.ds(i, 1), :]
        return keep * (1.0 - row * k_i)

    keep = lax.fori_loop(0, _NMS_PRE, body, jnp.ones((1, _NC), jnp.float32))
    keepb = keep * jnp.where(scores_r > _SCORE_THR, 1.0, 0.0)
    gi = lax.broadcasted_iota(jnp.int32, (_NC, _NC), 0)
    gj = lax.broadcasted_iota(jnp.int32, (_NC, _NC), 1)
    ut = jnp.where(gi <= gj, 1.0, 0.0)
    ident = jnp.where(gi == gj, 1.0, 0.0)
    c1 = jax.lax.dot_general(keepb, ut, (((1,), (0,)), ((), ())),
                             preferred_element_type=jnp.float32)
    cn1 = jax.lax.dot_general(1.0 - keepb, ut, (((1,), (0,)), ((), ())),
                              preferred_element_type=jnp.float32)
    K = jnp.sum(keepb)
    pos = jnp.where(keepb > 0.0, c1 - 1.0, K + cn1 - 1.0)
    keepb_c = jax.lax.dot_general(ident, keepb, (((1,), (1,)), ((), ())),
                                  preferred_element_type=jnp.float32)
    p = jnp.where(
        lax.broadcasted_iota(jnp.int32, (128, _NC), 0).astype(jnp.float32)
        == pos, 1.0, 0.0)

    def sel(col):
        return jax.lax.dot_general(p, col, (((1,), (0,)), ((), ())),
                                   preferred_element_type=jnp.float32)

    x1o[...] = sel(boxes_c[:, 0:1])
    y1o[...] = sel(boxes_c[:, 1:2])
    x2o[...] = sel(boxes_c[:, 2:3])
    y2o[...] = sel(boxes_c[:, 3:4])
    so[...] = sel(jnp.where(keepb_c > 0.0, scores_c, 0.0))
    lo[...] = sel(labels_c)


def _nms_pipeline(cand_scores, cand_rois, cand_deltas, cand_labels, H, W):
    hw = jnp.stack([H.astype(jnp.float32), W.astype(jnp.float32)])
    scores_r = cand_scores.reshape(1, _NC)
    scores_c = cand_scores.reshape(_NC, 1)
    labels_f = cand_labels.astype(jnp.float32)
    labels_r = labels_f.reshape(1, _NC)
    labels_c = labels_f.reshape(_NC, 1)
    rois_r = cand_rois.T
    deltas_r = cand_deltas.T

    s_mat, boxes_c = pl.pallas_call(
        _smat_kernel,
        grid=(_NC // 128,),
        in_specs=[
            pl.BlockSpec(memory_space=pltpu.SMEM),
            pl.BlockSpec((128, 4), lambda i: (i, 0)),
            pl.BlockSpec((128, 4), lambda i: (i, 0)),
            pl.BlockSpec((128, 1), lambda i: (i, 0)),
            pl.BlockSpec((4, _NC), lambda i: (0, 0)),
            pl.BlockSpec((4, _NC), lambda i: (0, 0)),
            pl.BlockSpec((1, _NC), lambda i: (0, 0)),
        ],
        out_specs=[
            pl.BlockSpec((128, _NC), lambda i: (i, 0)),
            pl.BlockSpec((128, 4), lambda i: (i, 0)),
        ],
        out_shape=[
            jax.ShapeDtypeStruct((_NC, _NC), jnp.float32),
            jax.ShapeDtypeStruct((_NC, 4), jnp.float32),
        ],
    )(hw, cand_rois, cand_deltas, labels_c, rois_r, deltas_r, labels_r)

    outs = pl.pallas_call(
        _nms_kernel,
        out_shape=[jax.ShapeDtypeStruct((128, 1), jnp.float32)] * 6,
    )(s_mat, boxes_c, scores_r, scores_c, labels_c)
    x1, y1, x2, y2, sc, lb = outs
    det_boxes = jnp.concatenate(
        [x1[:_MAX_PER_IMG], y1[:_MAX_PER_IMG], x2[:_MAX_PER_IMG],
         y2[:_MAX_PER_IMG]], axis=1)
    det_scores = sc[:_MAX_PER_IMG, 0]
    det_labels = jnp.round(lb[:_MAX_PER_IMG, 0]).astype(jnp.int32)
    return det_boxes, det_scores, det_labels


def kernel(class_outs, regression_outs, boxes, image_h, image_w):
    H = jnp.asarray(image_h, jnp.float32)
    W = jnp.asarray(image_w, jnp.float32)
    scores = _softmax_scores(class_outs)
    flat = scores.reshape(-1)
    masked = jnp.where(flat > _SCORE_THR, flat, -jnp.inf)
    top_scores, idx = lax.top_k(masked, _NMS_PRE)
    prop = idx // _N_CLS
    cls = idx % _N_CLS
    cand_rois = boxes[prop, :4]
    cand_deltas = regression_outs.reshape(_N_PROP, _N_CLS, 4)[prop, cls]
    pad = _NC - _NMS_PRE
    cand_scores = jnp.concatenate(
        [top_scores, jnp.full((pad,), -jnp.inf, jnp.float32)])
    cand_rois = jnp.concatenate(
        [cand_rois, jnp.zeros((pad, 4), jnp.float32)])
    cand_deltas = jnp.concatenate(
        [cand_deltas, jnp.zeros((pad, 4), jnp.float32)])
    cand_labels = jnp.concatenate(
        [cls, jnp.full((pad,), _N_CLS + 20, jnp.int32)])
    return _nms_pipeline(cand_scores, cand_rois, cand_deltas, cand_labels,
                         H, W)


# speedup vs baseline: 8.8264x; 5.7217x over previous
import functools

import jax
import jax.numpy as jnp
from jax import lax
from jax.experimental import pallas as pl
from jax.experimental.pallas import tpu as pltpu
from jax.experimental.pallas import tpu_sc as plsc

_N_PROP = 20000
_N_CLS = 80
_NFLAT = _N_PROP * _N_CLS
_SCORE_THR = 0.05
_IOU_THR = 0.5
_MAX_PER_IMG = 100
_NMS_PRE = 1000
_MAX_RATIO = 4.135166556742356
_OFFSET = 4096.0
_NC = 1024
_NW = 32
_PER_W = _NFLAT // _NW
_SLAB = 2048
_CBUF = 2048
_BIGIDX = 0x3FFFFFFF
_NEGINF_BITS = -8388608


def _softmax_kernel(x_ref, o_ref):
    x = x_ref[...]
    m = jnp.max(x, axis=1, keepdims=True)
    e = jnp.exp(x - m)
    s = jnp.sum(e, axis=1, keepdims=True)
    o_ref[...] = e[:, :_N_CLS] / s


def _softmax_scores(class_outs):
    blk = 2000
    return pl.pallas_call(
        _softmax_kernel,
        grid=(_N_PROP // blk,),
        in_specs=[pl.BlockSpec((blk, _N_CLS + 1), lambda i: (i, 0))],
        out_specs=pl.BlockSpec((blk, _N_CLS), lambda i: (i, 0)),
        out_shape=jax.ShapeDtypeStruct((_N_PROP, _N_CLS), jnp.float32),
    )(class_outs)


def _sc_mesh():
    return plsc.VectorSubcoreMesh(core_axis_name="c", subcore_axis_name="s")


def _compact1(scores_flat):
    @functools.partial(
        pl.kernel,
        mesh=_sc_mesh(),
        compiler_params=pltpu.CompilerParams(needs_layout_passes=False),
        out_type=[
            jax.ShapeDtypeStruct((_NW, _SLAB), jnp.int32),
            jax.ShapeDtypeStruct((_NW, _SLAB), jnp.int32),
        ],
        scratch_types=[
            pltpu.VMEM((_PER_W,), jnp.float32),
            pltpu.VMEM((_SLAB,), jnp.int32),
            pltpu.VMEM((_SLAB,), jnp.int32),
        ],
    )
    def k(s_hbm, slab_s_hbm, slab_i_hbm, buf, sbuf, ibuf):
        wid = lax.axis_index("s") * 2 + lax.axis_index("c")
        base = wid * _PER_W
        pltpu.sync_copy(s_hbm.at[pl.ds(base, _PER_W)], buf)
        neg = jnp.full((16,), _NEGINF_BITS, jnp.int32)
        bigi = jnp.full((16,), _BIGIDX, jnp.int32)

        def fill(i, carry):
            sbuf[pl.ds(i * 16, 16)] = neg
            ibuf[pl.ds(i * 16, 16)] = bigi
            return carry

        lax.fori_loop(0, _SLAB // 16, fill, 0)
        iota16 = lax.iota(jnp.int32, 16)

        def step(i, off):
            v = buf[pl.ds(i * 16, 16)]
            mask = v > _SCORE_THR
            c = jnp.cumsum(jnp.where(mask, 1, 0))
            pos = jnp.minimum(off + c - 1, _SLAB - 1)
            plsc.store_scatter(sbuf, [pos], plsc.bitcast(v, jnp.int32),
                               mask=mask)
            idxv = iota16 + (base + i * 16)
            plsc.store_scatter(ibuf, [pos], idxv, mask=mask)
            return off + jnp.max(c)

        lax.fori_loop(0, _PER_W // 16, step, jnp.int32(0))
        pltpu.sync_copy(sbuf, slab_s_hbm.at[wid])
        pltpu.sync_copy(ibuf, slab_i_hbm.at[wid])

    return k(scores_flat)


def _thresh_kernel(s_ref, t_ref, off_ref):
    s = s_ref[...]

    def it(_, lohi):
        lo, hi = lohi
        mid = (lo + hi) * 0.5
        cnt = jnp.sum(jnp.where(s > mid, 1.0, 0.0))
        pred = cnt >= float(_NMS_PRE)
        return (jnp.where(pred, mid, lo), jnp.where(pred, hi, mid))

    lo, _ = lax.fori_loop(0, 30, it, (jnp.float32(_SCORE_THR),
                                      jnp.float32(1.0)))
    t_ref[...] = jnp.full((1, 16), lo, jnp.float32)
    cnt_rows = jnp.sum(jnp.where(s > lo, 1.0, 0.0), axis=1, keepdims=True)
    pad_rows = jnp.floor((cnt_rows + 15.0) * (1.0 / 16.0)) * 16.0
    gi = lax.broadcasted_iota(jnp.int32, (_NW + 1, _NW), 0)
    gj = lax.broadcasted_iota(jnp.int32, (_NW + 1, _NW), 1)
    tri = jnp.where(gj < gi, 1.0, 0.0)
    goff = jax.lax.dot_general(tri, pad_rows, (((1,), (0,)), ((), ())),
                               preferred_element_type=jnp.float32,
                             precision=jax.lax.Precision.HIGHEST)
    off_ref[...] = jnp.broadcast_to(goff, (_NW + 1, 16)).astype(jnp.int32)


def _find_threshold(slab_s):
    tlow, offs = pl.pallas_call(
        _thresh_kernel,
        out_shape=[
            jax.ShapeDtypeStruct((1, 16), jnp.float32),
            jax.ShapeDtypeStruct((_NW + 1, 16), jnp.int32),
        ],
    )(slab_s)
    return tlow.reshape(-1), offs


def _compact2(slab_s, slab_i, tlow16, offs):
    @functools.partial(
        pl.kernel,
        mesh=_sc_mesh(),
        compiler_params=pltpu.CompilerParams(needs_layout_passes=False),
        out_type=[
            jax.ShapeDtypeStruct((_CBUF,), jnp.int32),
            jax.ShapeDtypeStruct((_CBUF,), jnp.int32),
        ],
        scratch_types=[
            pltpu.VMEM((_SLAB,), jnp.int32),
            pltpu.VMEM((_SLAB,), jnp.int32),
            pltpu.VMEM((16,), jnp.float32),
            pltpu.VMEM((_SLAB,), jnp.int32),
            pltpu.VMEM((_SLAB,), jnp.int32),
            pltpu.VMEM((16,), jnp.int32),
            pltpu.VMEM((16,), jnp.int32),
        ],
    )
    def k(ss_hbm, si_hbm, tl_hbm, off_hbm, cs_hbm, ci_hbm,
          sbuf, ibuf, tbuf, osbuf, oibuf, goffb, nextb):
        wid = lax.axis_index("s") * 2 + lax.axis_index("c")
        pltpu.sync_copy(ss_hbm.at[wid], sbuf)
        pltpu.sync_copy(si_hbm.at[wid], ibuf)
        pltpu.sync_copy(tl_hbm, tbuf)
        pltpu.sync_copy(off_hbm.at[wid], goffb)
        pltpu.sync_copy(off_hbm.at[wid + 1], nextb)
        tv = tbuf[...]
        goff = jnp.max(goffb[...])
        pad16 = jnp.max(nextb[...]) - goff
        neg = jnp.full((16,), _NEGINF_BITS, jnp.int32)
        bigi = jnp.full((16,), _BIGIDX, jnp.int32)

        def fill(i, carry):
            osbuf[pl.ds(i * 16, 16)] = neg
            oibuf[pl.ds(i * 16, 16)] = bigi
            return carry

        lax.fori_loop(0, _SLAB // 16, fill, 0)

        def step(i, off):
            vi = sbuf[pl.ds(i * 16, 16)]
            iv = ibuf[pl.ds(i * 16, 16)]
            mask = plsc.bitcast(vi, jnp.float32) > tv
            c = jnp.cumsum(jnp.where(mask, 1, 0))
            pos = jnp.minimum(off + c - 1, _SLAB - 1)
            plsc.store_scatter(osbuf, [pos], vi, mask=mask)
            plsc.store_scatter(oibuf, [pos], iv, mask=mask)
            return off + jnp.max(c)

        lax.fori_loop(0, _SLAB // 16, step, jnp.int32(0))

        def emit(kk, carry):
            dst = pl.multiple_of(jnp.minimum(goff + kk * 16, _CBUF - 16), 16)
            pltpu.sync_copy(osbuf.at[pl.ds(kk * 16, 16)],
                            cs_hbm.at[pl.ds(dst, 16)])
            pltpu.sync_copy(oibuf.at[pl.ds(kk * 16, 16)],
                            ci_hbm.at[pl.ds(dst, 16)])
            return carry

        lax.fori_loop(0, pad16 // 16, emit, 0)

    return k(slab_s, slab_i, tlow16, offs)


def _rank_kernel(mp_ref, scol_ref, icol_ref, srow_ref, irow_ref, rank_ref):
    mp = mp_ref[0]
    pid = pl.program_id(0)
    blk = scol_ref.shape[0]
    gcol = pid * blk + lax.broadcasted_iota(jnp.int32, (blk, 1), 0)
    grow = lax.broadcasted_iota(jnp.int32, (1, _CBUF), 1)
    scol = jnp.where(gcol < mp, scol_ref[...], -jnp.inf)
    icol = jnp.where(gcol < mp, icol_ref[...], _BIGIDX)
    srow = jnp.where(grow < mp, srow_ref[...], -jnp.inf)
    irow = jnp.where(grow < mp, irow_ref[...], _BIGIDX)
    cmp = (srow > scol) | ((srow == scol) & (irow < icol))
    rank_ref[...] = jnp.sum(jnp.where(cmp, 1.0, 0.0), axis=1, keepdims=True)


def _select_kernel(rank_ref, feat_ref, o_ref):
    pid = pl.program_id(0)

    @pl.when(pid == 0)
    def _():
        o_ref[...] = jnp.zeros_like(o_ref)

    blk = feat_ref.shape[0]
    rank = rank_ref[...]
    riota = lax.broadcasted_iota(jnp.int32, (_NC, blk), 0).astype(jnp.float32)
    p = jnp.where(riota == rank, 1.0, 0.0)
    o_ref[...] += jax.lax.dot_general(
        p, feat_ref[...], (((1,), (0,)), ((), ())),
        preferred_element_type=jnp.float32,
                             precision=jax.lax.Precision.HIGHEST)


def _top_candidates(cand_s, cand_i, mp16):
    blk = 512
    scol = cand_s.reshape(_CBUF, 1)
    srow = cand_s.reshape(1, _CBUF)
    icol = cand_i.reshape(_CBUF, 1)
    irow = cand_i.reshape(1, _CBUF)
    rank = pl.pallas_call(
        _rank_kernel,
        grid=(_CBUF // blk,),
        in_specs=[
            pl.BlockSpec(memory_space=pltpu.SMEM),
            pl.BlockSpec((blk, 1), lambda i: (i, 0)),
            pl.BlockSpec((blk, 1), lambda i: (i, 0)),
            pl.BlockSpec((1, _CBUF), lambda i: (0, 0)),
            pl.BlockSpec((1, _CBUF), lambda i: (0, 0)),
        ],
        out_specs=pl.BlockSpec((blk, 1), lambda i: (i, 0)),
        out_shape=jax.ShapeDtypeStruct((_CBUF, 1), jnp.float32),
    )(mp16, scol, icol, srow, irow)

    feat = jnp.concatenate(
        [cand_s.reshape(-1, 1), cand_i.astype(jnp.float32).reshape(-1, 1),
         jnp.zeros((_CBUF, 6), jnp.float32)], axis=1)
    out = pl.pallas_call(
        _select_kernel,
        grid=(_CBUF // blk,),
        in_specs=[
            pl.BlockSpec((1, blk), lambda i: (0, i)),
            pl.BlockSpec((blk, 8), lambda i: (i, 0)),
        ],
        out_specs=pl.BlockSpec((_NC, 8), lambda i: (0, 0)),
        out_shape=jax.ShapeDtypeStruct((_NC, 8), jnp.float32),
    )(rank.reshape(1, _CBUF), feat)
    return out[:, 0], out[:, 1]


def _gather_cands(reg128, box128, idx1024):
    per_w = _NC // _NW

    @functools.partial(
        pl.kernel,
        mesh=_sc_mesh(),
        compiler_params=pltpu.CompilerParams(needs_layout_passes=False),
        out_type=[
            jax.ShapeDtypeStruct((_NC, 4), jnp.float32),
            jax.ShapeDtypeStruct((_NC, 4), jnp.float32),
        ],
        scratch_types=[
            pltpu.VMEM((per_w,), jnp.int32),
            pltpu.VMEM((per_w,), jnp.int32),
            pltpu.VMEM((per_w,), jnp.int32),
            pltpu.VMEM((per_w, 128), jnp.float32),
            pltpu.VMEM((per_w, 128), jnp.float32),
            pltpu.VMEM((per_w, 4), jnp.float32),
            pltpu.VMEM((per_w, 4), jnp.float32),
            pltpu.SemaphoreType.DMA,
        ],
    )
    def k(reg_hbm, box_hbm, idx_hbm, od_hbm, or_hbm,
          idxb, drowi, browi, dbuf, bbuf, odbuf, orbuf, sem):
        wid = lax.axis_index("s") * 2 + lax.axis_index("c")
        base = wid * per_w
        pltpu.sync_copy(idx_hbm.at[pl.ds(base, per_w)], idxb)
        iota16 = lax.iota(jnp.int32, 16)
        for j in range(per_w // 16):
            iv = idxb[pl.ds(j * 16, 16)]
            drowi[pl.ds(j * 16, 16)] = iv // 32
            browi[pl.ds(j * 16, 16)] = (iv // _N_CLS) // 32
        pltpu.async_copy(reg_hbm.at[drowi], dbuf, sem).wait()
        pltpu.async_copy(box_hbm.at[browi], bbuf, sem).wait()
        for j in range(per_w // 16):
            iv = idxb[pl.ds(j * 16, 16)]
            lrow = j * 16 + iota16
            dcol = 4 * (iv % 32)
            pv = iv // _N_CLS
            bcol = 4 * (pv % 32)
            for c in range(4):
                cc = jnp.full((16,), c, jnp.int32)
                dv = plsc.load_gather(dbuf, [lrow, dcol + c])
                plsc.store_scatter(odbuf, [lrow, cc], dv)
                bv = plsc.load_gather(bbuf, [lrow, bcol + c])
                plsc.store_scatter(orbuf, [lrow, cc], bv)
        pltpu.sync_copy(odbuf, od_hbm.at[pl.ds(base, per_w)])
        pltpu.sync_copy(orbuf, or_hbm.at[pl.ds(base, per_w)])

    return k(reg128, box128, idx1024)


def _decode_rc(rois, deltas, labels, W, H):
    dx = deltas[0] * 0.1
    dy = deltas[1] * 0.1
    dw = jnp.clip(deltas[2] * 0.2, -_MAX_RATIO, _MAX_RATIO)
    dh = jnp.clip(deltas[3] * 0.2, -_MAX_RATIO, _MAX_RATIO)
    pw = rois[2] - rois[0]
    ph = rois[3] - rois[1]
    px = (rois[0] + rois[2]) * 0.5
    py = (rois[1] + rois[3]) * 0.5
    gw = pw * jnp.exp(dw)
    gh = ph * jnp.exp(dh)
    gx = px + pw * dx
    gy = py + ph * dy
    x1 = jnp.clip(gx - gw * 0.5, 0.0, W)
    y1 = jnp.clip(gy - gh * 0.5, 0.0, H)
    x2 = jnp.clip(gx + gw * 0.5, 0.0, W)
    y2 = jnp.clip(gy + gh * 0.5, 0.0, H)
    off = labels * _OFFSET
    return (x1, y1, x2, y2), (x1 + off, y1 + off, x2 + off, y2 + off)


def _smat_kernel(hw_ref, rois_c, deltas_c, labels_c, rois_r, deltas_r,
                 labels_r, s_ref, boxes_c_ref):
    H = hw_ref[0]
    W = hw_ref[1]
    pid = pl.program_id(0)
    roc = rois_c[...]
    dec = deltas_c[...]
    rc = [roc[:, k:k + 1] for k in range(4)]
    dc = [dec[:, k:k + 1] for k in range(4)]
    (x1c, y1c, x2c, y2c), (ox1c, oy1c, ox2c, oy2c) = _decode_rc(
        rc, dc, labels_c[...], W, H)
    boxes_c_ref[:, 0:1] = x1c
    boxes_c_ref[:, 1:2] = y1c
    boxes_c_ref[:, 2:3] = x2c
    boxes_c_ref[:, 3:4] = y2c
    ror = rois_r[...]
    der = deltas_r[...]
    rr = [ror[k:k + 1, :] for k in range(4)]
    dr = [der[k:k + 1, :] for k in range(4)]
    _, (ox1r, oy1r, ox2r, oy2r) = _decode_rc(rr, dr, labels_r[...], W, H)
    area_c = jnp.maximum(ox2c - ox1c, 0.0) * jnp.maximum(oy2c - oy1c, 0.0)
    area_r = jnp.maximum(ox2r - ox1r, 0.0) * jnp.maximum(oy2r - oy1r, 0.0)
    ix1 = jnp.maximum(ox1c, ox1r)
    iy1 = jnp.maximum(oy1c, oy1r)
    ix2 = jnp.minimum(ox2c, ox2r)
    iy2 = jnp.minimum(oy2c, oy2r)
    inter = jnp.maximum(ix2 - ix1, 0.0) * jnp.maximum(iy2 - iy1, 0.0)
    iou = inter / (area_c + area_r - inter + 1e-6)
    gi = pid * 128 + lax.broadcasted_iota(jnp.int32, (128, _NC), 0)
    gj = lax.broadcasted_iota(jnp.int32, (128, _NC), 1)
    s_ref[...] = jnp.where((iou > _IOU_THR) & (gi < gj), 1.0, 0.0)


def _nms_kernel(s_ref, boxes_c_ref, scores_r_ref, scores_c_ref, labels_c_ref,
                x1o, y1o, x2o, y2o, so, lo):
    boxes_c = boxes_c_ref[...]
    scores_r = scores_r_ref[...]
    scores_c = scores_c_ref[...]
    labels_c = labels_c_ref[...]
    iota_r = lax.broadcasted_iota(jnp.int32, (1, _NC), 1)

    def body(i, keep):
        k_i = jnp.sum(jnp.where(iota_r == i, keep, 0.0))
        row = s_ref[pl.ds(i, 1), :]
        return keep * (1.0 - row * k_i)

    keep = lax.fori_loop(0, _NMS_PRE, body, jnp.ones((1, _NC), jnp.float32))
    valid = (scores_r > _SCORE_THR) & (iota_r < _NMS_PRE)
    keepb = keep * jnp.where(valid, 1.0, 0.0)
    gi = lax.broadcasted_iota(jnp.int32, (_NC, _NC), 0)
    gj = lax.broadcasted_iota(jnp.int32, (_NC, _NC), 1)
    ut = jnp.where(gi <= gj, 1.0, 0.0)
    ident = jnp.where(gi == gj, 1.0, 0.0)
    c1 = jax.lax.dot_general(keepb, ut, (((1,), (0,)), ((), ())),
                             preferred_element_type=jnp.float32,
                             precision=jax.lax.Precision.HIGHEST)
    cn1 = jax.lax.dot_general(1.0 - keepb, ut, (((1,), (0,)), ((), ())),
                              preferred_element_type=jnp.float32,
                             precision=jax.lax.Precision.HIGHEST)
    K = jnp.sum(keepb)
    pos = jnp.where(keepb > 0.0, c1 - 1.0, K + cn1 - 1.0)
    keepb_c = jax.lax.dot_general(ident, keepb, (((1,), (1,)), ((), ())),
                                  preferred_element_type=jnp.float32,
                             precision=jax.lax.Precision.HIGHEST)
    p = jnp.where(
        lax.broadcasted_iota(jnp.int32, (128, _NC), 0).astype(jnp.float32)
        == pos, 1.0, 0.0)

    def sel(col):
        return jax.lax.dot_general(p, col, (((1,), (0,)), ((), ())),
                                   preferred_element_type=jnp.float32,
                             precision=jax.lax.Precision.HIGHEST)

    x1o[...] = sel(boxes_c[:, 0:1])
    y1o[...] = sel(boxes_c[:, 1:2])
    x2o[...] = sel(boxes_c[:, 2:3])
    y2o[...] = sel(boxes_c[:, 3:4])
    so[...] = sel(jnp.where(keepb_c > 0.0, scores_c, 0.0))
    lo[...] = sel(labels_c)


def _nms_pipeline(cand_scores, cand_rois, cand_deltas, cand_labels, H, W):
    hw = jnp.stack([H.astype(jnp.float32), W.astype(jnp.float32)])
    scores_r = cand_scores.reshape(1, _NC)
    scores_c = cand_scores.reshape(_NC, 1)
    labels_f = cand_labels.astype(jnp.float32)
    labels_r = labels_f.reshape(1, _NC)
    labels_c = labels_f.reshape(_NC, 1)
    rois_r = cand_rois.T
    deltas_r = cand_deltas.T

    s_mat, boxes_c = pl.pallas_call(
        _smat_kernel,
        grid=(_NC // 128,),
        in_specs=[
            pl.BlockSpec(memory_space=pltpu.SMEM),
            pl.BlockSpec((128, 4), lambda i: (i, 0)),
            pl.BlockSpec((128, 4), lambda i: (i, 0)),
            pl.BlockSpec((128, 1), lambda i: (i, 0)),
            pl.BlockSpec((4, _NC), lambda i: (0, 0)),
            pl.BlockSpec((4, _NC), lambda i: (0, 0)),
            pl.BlockSpec((1, _NC), lambda i: (0, 0)),
        ],
        out_specs=[
            pl.BlockSpec((128, _NC), lambda i: (i, 0)),
            pl.BlockSpec((128, 4), lambda i: (i, 0)),
        ],
        out_shape=[
            jax.ShapeDtypeStruct((_NC, _NC), jnp.float32),
            jax.ShapeDtypeStruct((_NC, 4), jnp.float32),
        ],
    )(hw, cand_rois, cand_deltas, labels_c, rois_r, deltas_r, labels_r)

    outs = pl.pallas_call(
        _nms_kernel,
        out_shape=[jax.ShapeDtypeStruct((128, 1), jnp.float32)] * 6,
    )(s_mat, boxes_c, scores_r, scores_c, labels_c)
    x1, y1, x2, y2, sc, lb = outs
    det_boxes = jnp.concatenate(
        [x1[:_MAX_PER_IMG], y1[:_MAX_PER_IMG], x2[:_MAX_PER_IMG],
         y2[:_MAX_PER_IMG]], axis=1)
    det_scores = sc[:_MAX_PER_IMG, 0]
    det_labels = jnp.round(lb[:_MAX_PER_IMG, 0]).astype(jnp.int32)
    return det_boxes, det_scores, det_labels


@jax.jit
def _kernel_impl(class_outs, regression_outs, boxes, image_h, image_w):
    H = jnp.asarray(image_h, jnp.float32)
    W = jnp.asarray(image_w, jnp.float32)
    scores = _softmax_scores(class_outs)
    slab_s_raw, slab_i = _compact1(scores.reshape(-1))
    slab_s = lax.bitcast_convert_type(slab_s_raw, jnp.float32)
    tlow16, offs = _find_threshold(slab_s)
    cand_s_raw, cand_i = _compact2(slab_s_raw, slab_i, tlow16, offs)
    cand_s = jnp.maximum(
        lax.bitcast_convert_type(cand_s_raw, jnp.float32), -1e30)
    mp_smem = offs[_NW:_NW + 1, 0]
    top_s, top_if = _top_candidates(cand_s, cand_i, mp_smem)
    idx1024 = jnp.clip(jnp.round(top_if).astype(jnp.int32), 0, _NFLAT - 1)
    cand_deltas, cand_rois = _gather_cands(
        regression_outs.reshape(_NFLAT * 4 // 128, 128),
        boxes.reshape(_N_PROP * 4 // 128, 128), idx1024)
    cand_labels = idx1024 % _N_CLS
    return _nms_pipeline(top_s, cand_rois, cand_deltas, cand_labels, H, W)


def kernel(class_outs, regression_outs, boxes, image_h, image_w):
    return _kernel_impl(class_outs, regression_outs, boxes, image_h, image_w)


# speedup vs baseline: 10.5998x; 1.2009x over previous
import functools

import jax
import jax.numpy as jnp
from jax import lax
from jax.experimental import pallas as pl
from jax.experimental.pallas import tpu as pltpu
from jax.experimental.pallas import tpu_sc as plsc

_N_PROP = 20000
_N_CLS = 80
_NFLAT = _N_PROP * _N_CLS
_SCORE_THR = 0.05
_IOU_THR = 0.5
_MAX_PER_IMG = 100
_NMS_PRE = 1000
_MAX_RATIO = 4.135166556742356
_OFFSET = 4096.0
_NC = 1024
_NW = 32
_PER_W = _NFLAT // _NW
_ROWS_W = 624
_ROWS_TAIL = _N_PROP - _NW * _ROWS_W
_REG_PAD = 512
_SLAB = 2048
_CBUF = 2048
_BIGIDX = 0x3FFFFFFF
_NEGINF_BITS = -8388608


def _softmax_kernel(x_ref, o_ref):
    x = x_ref[...]
    m = jnp.max(x, axis=1, keepdims=True)
    e = jnp.exp(x - m)
    s = jnp.sum(e, axis=1, keepdims=True)
    o_ref[...] = e[:, :_N_CLS] / s


def _softmax_scores(class_outs):
    blk = 2000
    return pl.pallas_call(
        _softmax_kernel,
        grid=(_N_PROP // blk,),
        in_specs=[pl.BlockSpec((blk, _N_CLS + 1), lambda i: (i, 0))],
        out_specs=pl.BlockSpec((blk, _N_CLS), lambda i: (i, 0)),
        out_shape=jax.ShapeDtypeStruct((_N_PROP, _N_CLS), jnp.float32),
    )(class_outs)


def _sc_mesh():
    return plsc.VectorSubcoreMesh(core_axis_name="c", subcore_axis_name="s")


def _compact1(scores_flat):
    @functools.partial(
        pl.kernel,
        mesh=_sc_mesh(),
        compiler_params=pltpu.CompilerParams(needs_layout_passes=False),
        out_type=[
            jax.ShapeDtypeStruct((_NW, _SLAB), jnp.int32),
            jax.ShapeDtypeStruct((_NW, _SLAB), jnp.int32),
        ],
        scratch_types=[
            pltpu.VMEM((_ROWS_W + _ROWS_TAIL, _N_CLS), jnp.float32),
            pltpu.VMEM((_SLAB,), jnp.int32),
            pltpu.VMEM((_SLAB,), jnp.int32),
        ],
    )
    def k(s_hbm, slab_s_hbm, slab_i_hbm, buf, sbuf, ibuf):
        wid = lax.axis_index("s") * 2 + lax.axis_index("c")
        row0 = pl.multiple_of(wid * _ROWS_W, 8)
        base = row0 * _N_CLS
        pltpu.sync_copy(s_hbm.at[pl.ds(row0, _ROWS_W)],
                        buf.at[pl.ds(0, _ROWS_W)])

        @pl.when(wid == _NW - 1)
        def _():
            pltpu.sync_copy(s_hbm.at[pl.ds(_NW * _ROWS_W, _ROWS_TAIL)],
                            buf.at[pl.ds(_ROWS_W, _ROWS_TAIL)])
        nrows = jnp.where(wid == _NW - 1, _ROWS_W + _ROWS_TAIL, _ROWS_W)
        neg = jnp.full((16,), _NEGINF_BITS, jnp.int32)
        bigi = jnp.full((16,), _BIGIDX, jnp.int32)

        def fill(i, carry):
            sbuf[pl.ds(i * 16, 16)] = neg
            ibuf[pl.ds(i * 16, 16)] = bigi
            return carry

        lax.fori_loop(0, _SLAB // 16, fill, 0)
        iota16 = lax.iota(jnp.int32, 16)

        def step(r, off):
            for cch in range(_N_CLS // 16):
                v = buf[r, pl.ds(cch * 16, 16)]
                mask = v > _SCORE_THR
                c = jnp.cumsum(jnp.where(mask, 1, 0))
                pos = jnp.minimum(off + c - 1, _SLAB - 1)
                plsc.store_scatter(sbuf, [pos], plsc.bitcast(v, jnp.int32),
                                   mask=mask)
                idxv = iota16 + (base + r * _N_CLS + cch * 16)
                plsc.store_scatter(ibuf, [pos], idxv, mask=mask)
                off = off + jnp.max(c)
            return off

        lax.fori_loop(0, nrows, step, jnp.int32(0))
        pltpu.sync_copy(sbuf, slab_s_hbm.at[wid])
        pltpu.sync_copy(ibuf, slab_i_hbm.at[wid])

    return k(scores_flat)


def _thresh_kernel(s_ref, t_ref, off_ref):
    s = s_ref[...]

    def it(_, lohi):
        lo, hi = lohi
        mid = (lo + hi) * 0.5
        cnt = jnp.sum(jnp.where(s > mid, 1.0, 0.0))
        pred = cnt >= float(_NMS_PRE)
        return (jnp.where(pred, mid, lo), jnp.where(pred, hi, mid))

    lo, _ = lax.fori_loop(0, 30, it, (jnp.float32(_SCORE_THR),
                                      jnp.float32(1.0)))
    t_ref[...] = jnp.full((1, 16), lo, jnp.float32)
    cnt_rows = jnp.sum(jnp.where(s > lo, 1.0, 0.0), axis=1, keepdims=True)
    pad_rows = jnp.floor((cnt_rows + 15.0) * (1.0 / 16.0)) * 16.0
    gi = lax.broadcasted_iota(jnp.int32, (_NW + 1, _NW), 0)
    gj = lax.broadcasted_iota(jnp.int32, (_NW + 1, _NW), 1)
    tri = jnp.where(gj < gi, 1.0, 0.0)
    goff = jax.lax.dot_general(tri, pad_rows, (((1,), (0,)), ((), ())),
                               preferred_element_type=jnp.float32,
                             precision=jax.lax.Precision.HIGHEST)
    off_ref[...] = jnp.broadcast_to(goff, (_NW + 1, 16)).astype(jnp.int32)


def _find_threshold(slab_s):
    tlow, offs = pl.pallas_call(
        _thresh_kernel,
        out_shape=[
            jax.ShapeDtypeStruct((1, 16), jnp.float32),
            jax.ShapeDtypeStruct((_NW + 1, 16), jnp.int32),
        ],
    )(slab_s)
    return tlow.reshape(-1), offs


def _compact2(slab_s, slab_i, tlow16, offs):
    @functools.partial(
        pl.kernel,
        mesh=_sc_mesh(),
        compiler_params=pltpu.CompilerParams(needs_layout_passes=False),
        out_type=[
            jax.ShapeDtypeStruct((_CBUF,), jnp.int32),
            jax.ShapeDtypeStruct((_CBUF,), jnp.int32),
        ],
        scratch_types=[
            pltpu.VMEM((_SLAB,), jnp.int32),
            pltpu.VMEM((_SLAB,), jnp.int32),
            pltpu.VMEM((16,), jnp.float32),
            pltpu.VMEM((_SLAB,), jnp.int32),
            pltpu.VMEM((_SLAB,), jnp.int32),
            pltpu.VMEM((16,), jnp.int32),
            pltpu.VMEM((16,), jnp.int32),
        ],
    )
    def k(ss_hbm, si_hbm, tl_hbm, off_hbm, cs_hbm, ci_hbm,
          sbuf, ibuf, tbuf, osbuf, oibuf, goffb, nextb):
        wid = lax.axis_index("s") * 2 + lax.axis_index("c")
        pltpu.sync_copy(ss_hbm.at[wid], sbuf)
        pltpu.sync_copy(si_hbm.at[wid], ibuf)
        pltpu.sync_copy(tl_hbm, tbuf)
        pltpu.sync_copy(off_hbm.at[wid], goffb)
        pltpu.sync_copy(off_hbm.at[wid + 1], nextb)
        tv = tbuf[...]
        goff = jnp.max(goffb[...])
        pad16 = jnp.max(nextb[...]) - goff
        neg = jnp.full((16,), _NEGINF_BITS, jnp.int32)
        bigi = jnp.full((16,), _BIGIDX, jnp.int32)

        def fill(i, carry):
            osbuf[pl.ds(i * 16, 16)] = neg
            oibuf[pl.ds(i * 16, 16)] = bigi
            return carry

        lax.fori_loop(0, _SLAB // 16, fill, 0)

        def step(i, off):
            vi = sbuf[pl.ds(i * 16, 16)]
            iv = ibuf[pl.ds(i * 16, 16)]
            mask = plsc.bitcast(vi, jnp.float32) > tv
            c = jnp.cumsum(jnp.where(mask, 1, 0))
            pos = jnp.minimum(off + c - 1, _SLAB - 1)
            plsc.store_scatter(osbuf, [pos], vi, mask=mask)
            plsc.store_scatter(oibuf, [pos], iv, mask=mask)
            return off + jnp.max(c)

        lax.fori_loop(0, _SLAB // 16, step, jnp.int32(0))

        def emit(kk, carry):
            dst = pl.multiple_of(jnp.minimum(goff + kk * 16, _CBUF - 16), 16)
            pltpu.sync_copy(osbuf.at[pl.ds(kk * 16, 16)],
                            cs_hbm.at[pl.ds(dst, 16)])
            pltpu.sync_copy(oibuf.at[pl.ds(kk * 16, 16)],
                            ci_hbm.at[pl.ds(dst, 16)])
            return carry

        lax.fori_loop(0, pad16 // 16, emit, 0)

    return k(slab_s, slab_i, tlow16, offs)


def _rank_kernel(mp_ref, scol_ref, icol_ref, srow_ref, irow_ref, rank_ref):
    mp = mp_ref[0]
    pid = pl.program_id(0)
    blk = scol_ref.shape[0]
    gcol = pid * blk + lax.broadcasted_iota(jnp.int32, (blk, 1), 0)
    grow = lax.broadcasted_iota(jnp.int32, (1, _CBUF), 1)
    scol = jnp.where(gcol < mp, scol_ref[...], -jnp.inf)
    icol = jnp.where(gcol < mp, icol_ref[...], _BIGIDX)
    srow = jnp.where(grow < mp, srow_ref[...], -jnp.inf)
    irow = jnp.where(grow < mp, irow_ref[...], _BIGIDX)
    cmp = (srow > scol) | ((srow == scol) & (irow < icol))
    rank_ref[...] = jnp.sum(jnp.where(cmp, 1.0, 0.0), axis=1, keepdims=True)


def _select_kernel(rank_ref, feat_ref, o_ref):
    pid = pl.program_id(0)

    @pl.when(pid == 0)
    def _():
        o_ref[...] = jnp.zeros_like(o_ref)

    blk = feat_ref.shape[0]
    rank = rank_ref[...]
    riota = lax.broadcasted_iota(jnp.int32, (_NC, blk), 0).astype(jnp.float32)
    p = jnp.where(riota == rank, 1.0, 0.0)
    o_ref[...] += jax.lax.dot_general(
        p, feat_ref[...], (((1,), (0,)), ((), ())),
        preferred_element_type=jnp.float32,
                             precision=jax.lax.Precision.HIGHEST)


def _top_candidates(cand_s, cand_i, mp16):
    blk = 512
    scol = cand_s.reshape(_CBUF, 1)
    srow = cand_s.reshape(1, _CBUF)
    icol = cand_i.reshape(_CBUF, 1)
    irow = cand_i.reshape(1, _CBUF)
    rank = pl.pallas_call(
        _rank_kernel,
        grid=(_CBUF // blk,),
        in_specs=[
            pl.BlockSpec(memory_space=pltpu.SMEM),
            pl.BlockSpec((blk, 1), lambda i: (i, 0)),
            pl.BlockSpec((blk, 1), lambda i: (i, 0)),
            pl.BlockSpec((1, _CBUF), lambda i: (0, 0)),
            pl.BlockSpec((1, _CBUF), lambda i: (0, 0)),
        ],
        out_specs=pl.BlockSpec((blk, 1), lambda i: (i, 0)),
        out_shape=jax.ShapeDtypeStruct((_CBUF, 1), jnp.float32),
    )(mp16, scol, icol, srow, irow)

    feat = jnp.concatenate(
        [cand_s.reshape(-1, 1), cand_i.astype(jnp.float32).reshape(-1, 1),
         jnp.zeros((_CBUF, 6), jnp.float32)], axis=1)
    out = pl.pallas_call(
        _select_kernel,
        grid=(_CBUF // blk,),
        in_specs=[
            pl.BlockSpec((1, blk), lambda i: (0, i)),
            pl.BlockSpec((blk, 8), lambda i: (i, 0)),
        ],
        out_specs=pl.BlockSpec((_NC, 8), lambda i: (0, 0)),
        out_shape=jax.ShapeDtypeStruct((_NC, 8), jnp.float32),
    )(rank.reshape(1, _CBUF), feat)
    return out[:, 0], out[:, 1]


def _pad_kernel(x_ref, o_ref):
    o_ref[:, 0:_N_CLS * 4] = x_ref[...]
    o_ref[:, _N_CLS * 4:_REG_PAD] = jnp.zeros(
        (x_ref.shape[0], _REG_PAD - _N_CLS * 4), jnp.float32)


def _pad_reg(reg):
    blk = 2000
    return pl.pallas_call(
        _pad_kernel,
        grid=(_N_PROP // blk,),
        in_specs=[pl.BlockSpec((blk, _N_CLS * 4), lambda i: (i, 0))],
        out_specs=pl.BlockSpec((blk, _REG_PAD), lambda i: (i, 0)),
        out_shape=jax.ShapeDtypeStruct((_N_PROP, _REG_PAD), jnp.float32),
    )(reg)


def _gather_cands(regp, box128, idx1024):
    per_w = _NC // _NW

    @functools.partial(
        pl.kernel,
        mesh=_sc_mesh(),
        compiler_params=pltpu.CompilerParams(needs_layout_passes=False),
        out_type=[
            jax.ShapeDtypeStruct((_NC, 4), jnp.float32),
            jax.ShapeDtypeStruct((_NC, 4), jnp.float32),
        ],
        scratch_types=[
            pltpu.VMEM((per_w,), jnp.int32),
            pltpu.VMEM((per_w,), jnp.int32),
            pltpu.VMEM((per_w,), jnp.int32),
            pltpu.VMEM((per_w, _REG_PAD), jnp.float32),
            pltpu.VMEM((per_w, 128), jnp.float32),
            pltpu.VMEM((per_w, 4), jnp.float32),
            pltpu.VMEM((per_w, 4), jnp.float32),
            pltpu.SemaphoreType.DMA,
        ],
    )
    def k(reg_hbm, box_hbm, idx_hbm, od_hbm, or_hbm,
          idxb, propb, browi, dbuf, bbuf, odbuf, orbuf, sem):
        wid = lax.axis_index("s") * 2 + lax.axis_index("c")
        base = wid * per_w
        pltpu.sync_copy(idx_hbm.at[pl.ds(base, per_w)], idxb)
        iota16 = lax.iota(jnp.int32, 16)
        for j in range(per_w // 16):
            iv = idxb[pl.ds(j * 16, 16)]
            pv = iv // _N_CLS
            propb[pl.ds(j * 16, 16)] = pv
            browi[pl.ds(j * 16, 16)] = pv // 32
        pltpu.async_copy(reg_hbm.at[propb], dbuf, sem).wait()
        pltpu.async_copy(box_hbm.at[browi], bbuf, sem).wait()
        for j in range(per_w // 16):
            iv = idxb[pl.ds(j * 16, 16)]
            lrow = j * 16 + iota16
            clsv = iv % _N_CLS
            pv = iv // _N_CLS
            bcol = 4 * (pv % 32)
            for c in range(4):
                cc = jnp.full((16,), c, jnp.int32)
                dv = plsc.load_gather(dbuf, [lrow, 4 * clsv + c])
                plsc.store_scatter(odbuf, [lrow, cc], dv)
                bv = plsc.load_gather(bbuf, [lrow, bcol + c])
                plsc.store_scatter(orbuf, [lrow, cc], bv)
        pltpu.sync_copy(odbuf, od_hbm.at[pl.ds(base, per_w)])
        pltpu.sync_copy(orbuf, or_hbm.at[pl.ds(base, per_w)])

    return k(regp, box128, idx1024)


def _decode_rc(rois, deltas, labels, W, H):
    dx = deltas[0] * 0.1
    dy = deltas[1] * 0.1
    dw = jnp.clip(deltas[2] * 0.2, -_MAX_RATIO, _MAX_RATIO)
    dh = jnp.clip(deltas[3] * 0.2, -_MAX_RATIO, _MAX_RATIO)
    pw = rois[2] - rois[0]
    ph = rois[3] - rois[1]
    px = (rois[0] + rois[2]) * 0.5
    py = (rois[1] + rois[3]) * 0.5
    gw = pw * jnp.exp(dw)
    gh = ph * jnp.exp(dh)
    gx = px + pw * dx
    gy = py + ph * dy
    x1 = jnp.clip(gx - gw * 0.5, 0.0, W)
    y1 = jnp.clip(gy - gh * 0.5, 0.0, H)
    x2 = jnp.clip(gx + gw * 0.5, 0.0, W)
    y2 = jnp.clip(gy + gh * 0.5, 0.0, H)
    off = labels * _OFFSET
    return (x1, y1, x2, y2), (x1 + off, y1 + off, x2 + off, y2 + off)


def _smat_kernel(hw_ref, rois_c, deltas_c, labels_c, rois_r, deltas_r,
                 labels_r, s_ref, boxes_c_ref):
    H = hw_ref[0]
    W = hw_ref[1]
    pid = pl.program_id(0)
    roc = rois_c[...]
    dec = deltas_c[...]
    rc = [roc[:, k:k + 1] for k in range(4)]
    dc = [dec[:, k:k + 1] for k in range(4)]
    (x1c, y1c, x2c, y2c), (ox1c, oy1c, ox2c, oy2c) = _decode_rc(
        rc, dc, labels_c[...], W, H)
    boxes_c_ref[:, 0:1] = x1c
    boxes_c_ref[:, 1:2] = y1c
    boxes_c_ref[:, 2:3] = x2c
    boxes_c_ref[:, 3:4] = y2c
    ror = rois_r[...]
    der = deltas_r[...]
    rr = [ror[k:k + 1, :] for k in range(4)]
    dr = [der[k:k + 1, :] for k in range(4)]
    _, (ox1r, oy1r, ox2r, oy2r) = _decode_rc(rr, dr, labels_r[...], W, H)
    area_c = jnp.maximum(ox2c - ox1c, 0.0) * jnp.maximum(oy2c - oy1c, 0.0)
    area_r = jnp.maximum(ox2r - ox1r, 0.0) * jnp.maximum(oy2r - oy1r, 0.0)
    ix1 = jnp.maximum(ox1c, ox1r)
    iy1 = jnp.maximum(oy1c, oy1r)
    ix2 = jnp.minimum(ox2c, ox2r)
    iy2 = jnp.minimum(oy2c, oy2r)
    inter = jnp.maximum(ix2 - ix1, 0.0) * jnp.maximum(iy2 - iy1, 0.0)
    iou = inter / (area_c + area_r - inter + 1e-6)
    gi = pid * 128 + lax.broadcasted_iota(jnp.int32, (128, _NC), 0)
    gj = lax.broadcasted_iota(jnp.int32, (128, _NC), 1)
    s_ref[...] = jnp.where((iou > _IOU_THR) & (gi < gj), 1.0, 0.0)


def _nms_kernel(s_ref, boxes_c_ref, scores_r_ref, scores_c_ref, labels_c_ref,
                x1o, y1o, x2o, y2o, so, lo):
    boxes_c = boxes_c_ref[...]
    scores_r = scores_r_ref[...]
    scores_c = scores_c_ref[...]
    labels_c = labels_c_ref[...]
    iota_r = lax.broadcasted_iota(jnp.int32, (1, _NC), 1)

    def body(i, keep):
        k_i = jnp.sum(jnp.where(iota_r == i, keep, 0.0))
        row = s_ref[pl.ds(i, 1), :]
        return keep * (1.0 - row * k_i)

    keep = lax.fori_loop(0, _NMS_PRE, body, jnp.ones((1, _NC), jnp.float32))
    valid = (scores_r > _SCORE_THR) & (iota_r < _NMS_PRE)
    keepb = keep * jnp.where(valid, 1.0, 0.0)
    gi = lax.broadcasted_iota(jnp.int32, (_NC, _NC), 0)
    gj = lax.broadcasted_iota(jnp.int32, (_NC, _NC), 1)
    ut = jnp.where(gi <= gj, 1.0, 0.0)
    ident = jnp.where(gi == gj, 1.0, 0.0)
    c1 = jax.lax.dot_general(keepb, ut, (((1,), (0,)), ((), ())),
                             preferred_element_type=jnp.float32,
                             precision=jax.lax.Precision.HIGHEST)
    cn1 = jax.lax.dot_general(1.0 - keepb, ut, (((1,), (0,)), ((), ())),
                              preferred_element_type=jnp.float32,
                             precision=jax.lax.Precision.HIGHEST)
    K = jnp.sum(keepb)
    pos = jnp.where(keepb > 0.0, c1 - 1.0, K + cn1 - 1.0)
    keepb_c = jax.lax.dot_general(ident, keepb, (((1,), (1,)), ((), ())),
                                  preferred_element_type=jnp.float32,
                             precision=jax.lax.Precision.HIGHEST)
    p = jnp.where(
        lax.broadcasted_iota(jnp.int32, (128, _NC), 0).astype(jnp.float32)
        == pos, 1.0, 0.0)

    def sel(col):
        return jax.lax.dot_general(p, col, (((1,), (0,)), ((), ())),
                                   preferred_element_type=jnp.float32,
                             precision=jax.lax.Precision.HIGHEST)

    x1o[...] = sel(boxes_c[:, 0:1])
    y1o[...] = sel(boxes_c[:, 1:2])
    x2o[...] = sel(boxes_c[:, 2:3])
    y2o[...] = sel(boxes_c[:, 3:4])
    so[...] = sel(jnp.where(keepb_c > 0.0, scores_c, 0.0))
    lo[...] = sel(labels_c)


def _nms_pipeline(cand_scores, cand_rois, cand_deltas, cand_labels, H, W):
    hw = jnp.stack([H.astype(jnp.float32), W.astype(jnp.float32)])
    scores_r = cand_scores.reshape(1, _NC)
    scores_c = cand_scores.reshape(_NC, 1)
    labels_f = cand_labels.astype(jnp.float32)
    labels_r = labels_f.reshape(1, _NC)
    labels_c = labels_f.reshape(_NC, 1)
    rois_r = cand_rois.T
    deltas_r = cand_deltas.T

    s_mat, boxes_c = pl.pallas_call(
        _smat_kernel,
        grid=(_NC // 128,),
        in_specs=[
            pl.BlockSpec(memory_space=pltpu.SMEM),
            pl.BlockSpec((128, 4), lambda i: (i, 0)),
            pl.BlockSpec((128, 4), lambda i: (i, 0)),
            pl.BlockSpec((128, 1), lambda i: (i, 0)),
            pl.BlockSpec((4, _NC), lambda i: (0, 0)),
            pl.BlockSpec((4, _NC), lambda i: (0, 0)),
            pl.BlockSpec((1, _NC), lambda i: (0, 0)),
        ],
        out_specs=[
            pl.BlockSpec((128, _NC), lambda i: (i, 0)),
            pl.BlockSpec((128, 4), lambda i: (i, 0)),
        ],
        out_shape=[
            jax.ShapeDtypeStruct((_NC, _NC), jnp.float32),
            jax.ShapeDtypeStruct((_NC, 4), jnp.float32),
        ],
    )(hw, cand_rois, cand_deltas, labels_c, rois_r, deltas_r, labels_r)

    outs = pl.pallas_call(
        _nms_kernel,
        out_shape=[jax.ShapeDtypeStruct((128, 1), jnp.float32)] * 6,
    )(s_mat, boxes_c, scores_r, scores_c, labels_c)
    x1, y1, x2, y2, sc, lb = outs
    det_boxes = jnp.concatenate(
        [x1[:_MAX_PER_IMG], y1[:_MAX_PER_IMG], x2[:_MAX_PER_IMG],
         y2[:_MAX_PER_IMG]], axis=1)
    det_scores = sc[:_MAX_PER_IMG, 0]
    det_labels = jnp.round(lb[:_MAX_PER_IMG, 0]).astype(jnp.int32)
    return det_boxes, det_scores, det_labels


@jax.jit
def _kernel_impl(class_outs, regression_outs, boxes, image_h, image_w):
    H = jnp.asarray(image_h, jnp.float32)
    W = jnp.asarray(image_w, jnp.float32)
    scores = _softmax_scores(class_outs)
    slab_s_raw, slab_i = _compact1(scores)
    slab_s = lax.bitcast_convert_type(slab_s_raw, jnp.float32)
    tlow16, offs = _find_threshold(slab_s)
    cand_s_raw, cand_i = _compact2(slab_s_raw, slab_i, tlow16, offs)
    cand_s = jnp.maximum(
        lax.bitcast_convert_type(cand_s_raw, jnp.float32), -1e30)
    mp_smem = offs[_NW:_NW + 1, 0]
    top_s, top_if = _top_candidates(cand_s, cand_i, mp_smem)
    idx1024 = jnp.clip(jnp.round(top_if).astype(jnp.int32), 0, _NFLAT - 1)
    cand_deltas, cand_rois = _gather_cands(
        _pad_reg(regression_outs), boxes.reshape(_N_PROP * 4 // 128, 128),
        idx1024)
    cand_labels = idx1024 % _N_CLS
    return _nms_pipeline(top_s, cand_rois, cand_deltas, cand_labels, H, W)


def kernel(class_outs, regression_outs, boxes, image_h, image_w):
    return _kernel_impl(class_outs, regression_outs, boxes, image_h, image_w)
